# speedup vs baseline: 1.2233x; 1.2233x over previous
from functools import partial

import jax
import jax.numpy as jnp
from jax.experimental import pallas as pl
from jax.experimental.pallas import tpu as pltpu


def _ghost_fused_kernel(x_ref, pww_ref, pws_ref, pwb_ref, cw_ref, cs_ref,
                        cb_ref, dww_ref, dws_ref, dwb_ref, o_ref,
                        x1p_ref, x2p_ref, *, H, W, C, Ho, Wo):
    xs = x_ref[0]
    wv = pww_ref[...]

    x1 = jax.lax.dot_general(xs, wv, (((0,), (0,)), ((), ())),
                             preferred_element_type=jnp.float32)
    x1 = x1 * pws_ref[...] + pwb_ref[...]
    x1 = jnp.maximum(x1, 0.0)
    x1 = x1.reshape(H, W, C)

    zrow = jnp.zeros((1, W + 2, C), jnp.float32)
    zcol = jnp.zeros((H + 2, 1, C), jnp.float32)
    for ref in (x1p_ref, x2p_ref):
        ref[0:1] = zrow
        ref[H + 1:H + 2] = zrow
        ref[:, 0:1] = zcol
        ref[:, W + 1:W + 2] = zcol

    x1p_ref[1:H + 1, 1:W + 1, :] = x1

    cwv = cw_ref[...]
    acc = jnp.zeros((H, W, C), jnp.float32)
    for ky in range(3):
        for kx in range(3):
            acc = acc + (x1p_ref[ky:ky + H, kx:kx + W, :]
                         * cwv[ky, kx].reshape(1, 1, C))
    x2 = acc * cs_ref[...].reshape(1, 1, C) + cb_ref[...].reshape(1, 1, C)
    x2 = jnp.maximum(x2, 0.0)
    x2p_ref[1:H + 1, 1:W + 1, :] = x2

    dwv = dww_ref[...]
    for half, src in ((0, x1p_ref), (1, x2p_ref)):
        sacc = jnp.zeros((Ho, Wo, C), jnp.float32)
        for ky in range(3):
            for kx in range(3):
                taps = src[pl.ds(ky, Ho, stride=2),
                           pl.ds(kx, Wo, stride=2), :]
                sacc = sacc + taps * dwv[ky, kx, half].reshape(1, 1, C)
        out = (sacc * dws_ref[half].reshape(1, 1, C)
               + dwb_ref[half].reshape(1, 1, C))
        o_ref[0, half] = out


def kernel(x_nchw, pw_w, pw_scale, pw_bias, cheap_w, cheap_scale, cheap_bias,
           dw_w, dw_scale, dw_bias):
    N, Cin, H, W = x_nchw.shape
    C = pw_w.shape[1]
    Ho = (H - 1) // 2 + 1
    Wo = (W - 1) // 2 + 1

    body = partial(_ghost_fused_kernel, H=H, W=W, C=C, Ho=Ho, Wo=Wo)
    out5 = pl.pallas_call(
        body,
        out_shape=jax.ShapeDtypeStruct((N, 2, Ho, Wo, C), jnp.float32),
        grid=(N,),
        in_specs=[
            pl.BlockSpec((1, Cin, H * W), lambda n: (n, 0, 0)),
            pl.BlockSpec((Cin, C), lambda n: (0, 0)),
            pl.BlockSpec((1, C), lambda n: (0, 0)),
            pl.BlockSpec((1, C), lambda n: (0, 0)),
            pl.BlockSpec((3, 3, C), lambda n: (0, 0, 0)),
            pl.BlockSpec((1, C), lambda n: (0, 0)),
            pl.BlockSpec((1, C), lambda n: (0, 0)),
            pl.BlockSpec((3, 3, 2, C), lambda n: (0, 0, 0, 0)),
            pl.BlockSpec((2, C), lambda n: (0, 0)),
            pl.BlockSpec((2, C), lambda n: (0, 0)),
        ],
        out_specs=pl.BlockSpec((1, 2, Ho, Wo, C), lambda n: (n, 0, 0, 0, 0)),
        scratch_shapes=[
            pltpu.VMEM((H + 2, W + 2, C), jnp.float32),
            pltpu.VMEM((H + 2, W + 2, C), jnp.float32),
        ],
        compiler_params=pltpu.CompilerParams(
            dimension_semantics=("parallel",),
            vmem_limit_bytes=64 * 1024 * 1024),
    )(x_nchw.reshape(N, Cin, H * W), pw_w,
      pw_scale.reshape(1, C), pw_bias.reshape(1, C),
      cheap_w, cheap_scale.reshape(1, C), cheap_bias.reshape(1, C),
      dw_w.reshape(3, 3, 2, C), dw_scale.reshape(2, C),
      dw_bias.reshape(2, C))

    return jnp.transpose(out5, (0, 1, 4, 2, 3)).reshape(N, 2 * C, Ho, Wo)
